# Initial kernel scaffold; baseline (speedup 1.0000x reference)
import jax
import jax.numpy as jnp
from jax.experimental import pallas as pl


def kernel(word_ids, lengths, embed_table, eof_embedding, unk_embedding, proj_w, proj_b):
    raise NotImplementedError("write your pallas kernel here")


# speedup vs baseline: 2.4792x; 2.4792x over previous
import functools

import jax
import jax.numpy as jnp
from jax import lax
from jax.experimental import pallas as pl
from jax.experimental.pallas import tpu as pltpu
from jax.experimental.pallas import tpu_sc as plsc

_B = 4096
_L = 51
_V = 100000
_D = 128
_DOUT = 128

_VPAD = _V + 8
_RBLK = 2000
_PGRID = _V // _RBLK + 1

_NC = 2
_NS = 16
_NW = _NC * _NS
_TOK = _B * _L
_TPW = _TOK // _NW
_RPW = _B // _NW
_CH = 128
_NCH = _TPW // _CH


def _proj_body(tab_ref, w_ref, b_ref, sp_ref, out_ref):
    w = w_ref[...]
    bias = b_ref[...]
    out_ref[...] = jnp.dot(tab_ref[...], w, preferred_element_type=jnp.float32) + bias

    @pl.when(pl.program_id(0) == _PGRID - 1)
    def _():
        out_ref[0:8, :] = jnp.dot(sp_ref[...], w, preferred_element_type=jnp.float32) + bias


_proj = pl.pallas_call(
    _proj_body,
    grid=(_PGRID,),
    in_specs=[
        pl.BlockSpec((_RBLK, _D), lambda i: (jnp.minimum(i, _V // _RBLK - 1), 0)),
        pl.BlockSpec((_D, _DOUT), lambda i: (0, 0)),
        pl.BlockSpec((1, _DOUT), lambda i: (0, 0)),
        pl.BlockSpec((8, _D), lambda i: (0, 0)),
    ],
    out_specs=pl.BlockSpec((_RBLK, _DOUT), lambda i: (i, 0)),
    out_shape=jax.ShapeDtypeStruct((_VPAD, _DOUT), jnp.float32),
)


def _gather_body(ptab, wid, lens, out, clen, ids_v, len_v, idx_v, buf, clen_v):
    c = lax.axis_index("c")
    s = lax.axis_index("s")
    w = s * _NC + c
    tbase = w * _TPW
    rbase = w * _RPW

    pltpu.sync_copy(wid.at[pl.ds(tbase, _TPW)], ids_v)
    pltpu.sync_copy(lens.at[pl.ds(rbase, _RPW)], len_v)

    def _clen_step(i, carry):
        clen_v[pl.ds(i * 16, 16)] = len_v[pl.ds(i * 16, 16)] + 1
        return carry

    lax.fori_loop(0, _RPW // 16, _clen_step, 0)
    pltpu.sync_copy(clen_v, clen.at[pl.ds(rbase, _RPW)])

    def _idx_step(i, carry):
        ids = ids_v[pl.ds(i * 16, 16)]
        g = tbase + i * 16 + lax.iota(jnp.int32, 16)
        b_loc = lax.div(g, jnp.int32(_L)) - rbase
        pos = lax.rem(g, jnp.int32(_L))
        lb = plsc.load_gather(len_v, [b_loc])
        row = jnp.where(ids < 0, _V + 1, jnp.where(pos == lb, _V, ids))
        idx_v[pl.ds(i * 16, 16)] = row
        return carry

    lax.fori_loop(0, _TPW // 16, _idx_step, 0)

    def _ch_step(k, carry):
        pltpu.sync_copy(ptab.at[idx_v.at[pl.ds(k * _CH, _CH)]], buf)
        pltpu.sync_copy(buf, out.at[pl.ds(tbase + k * _CH, _CH)])
        return carry

    lax.fori_loop(0, _NCH, _ch_step, 0)


_gather = pl.kernel(
    _gather_body,
    mesh=plsc.VectorSubcoreMesh(core_axis_name="c", subcore_axis_name="s"),
    compiler_params=pltpu.CompilerParams(needs_layout_passes=False),
    out_type=[
        jax.ShapeDtypeStruct((_TOK, _DOUT), jnp.float32),
        jax.ShapeDtypeStruct((_B,), jnp.int32),
    ],
    scratch_types=[
        pltpu.VMEM((_TPW,), jnp.int32),
        pltpu.VMEM((_RPW,), jnp.int32),
        pltpu.VMEM((_TPW,), jnp.int32),
        pltpu.VMEM((_CH, _DOUT), jnp.float32),
        pltpu.VMEM((_RPW,), jnp.int32),
    ],
)


def kernel(word_ids, lengths, embed_table, eof_embedding, unk_embedding, proj_w, proj_b):
    specials = jnp.concatenate(
        [eof_embedding, unk_embedding, jnp.zeros((6, _D), jnp.float32)], axis=0)
    ptab = _proj(embed_table, proj_w, proj_b.reshape(1, _DOUT), specials)
    out_flat, char_len = _gather(ptab, word_ids.reshape(-1), lengths)
    return out_flat.reshape(_B, _L, _DOUT), char_len
